# Initial kernel scaffold; baseline (speedup 1.0000x reference)
import jax
import jax.numpy as jnp
from jax.experimental import pallas as pl


def kernel(prev_hidden_states, encoder_output, tree_sizes, W, b, v):
    raise NotImplementedError("write your pallas kernel here")


# speedup vs baseline: 3.4442x; 3.4442x over previous
import jax
import jax.numpy as jnp
from jax import lax
from jax.experimental import pallas as pl
from jax.experimental.pallas import tpu as pltpu

_B = 256
_HE = 1024
_HD = 1024
_N = 32640
_T = 384


def _p_kernel(prev_ref, w1t_ref, b_ref, out_ref):
    out_ref[...] = (
        jnp.dot(prev_ref[...], w1t_ref[...], preferred_element_type=jnp.float32)
        + b_ref[...]
    )


def _scores_kernel(starts_ref, ends_ref, enc_ref, w2t_ref, p_ref, v_ref,
                   out_ref, mx_ref):
    t = pl.program_id(0)

    @pl.when(t == 0)
    def _init():
        mx_ref[...] = jnp.full((1, _B), -jnp.inf, jnp.float32)

    rows = t * _T + lax.broadcasted_iota(jnp.int32, (_T, 1), 0)
    in_seg = (rows >= starts_ref[...]) & (rows < ends_ref[...])
    oh = in_seg.astype(jnp.float32)
    pre = jnp.dot(enc_ref[...], w2t_ref[...], preferred_element_type=jnp.float32)
    pre = pre + jnp.dot(oh, p_ref[...], preferred_element_type=jnp.float32)
    energy = jnp.tanh(pre)
    s = jnp.sum(energy * v_ref[...], axis=1, keepdims=True)
    out_ref[...] = s
    vals = jnp.where(in_seg, s, -jnp.inf)
    mx_ref[...] = jnp.maximum(mx_ref[...], jnp.max(vals, axis=0, keepdims=True))


def _expsum_kernel(starts_ref, ends_ref, s_ref, mx_ref, out_ref):
    t = pl.program_id(0)

    @pl.when(t == 0)
    def _init():
        out_ref[...] = jnp.zeros((1, _B), jnp.float32)

    rows = t * _T + lax.broadcasted_iota(jnp.int32, (_T, 1), 0)
    in_seg = (rows >= starts_ref[...]) & (rows < ends_ref[...])
    mrow = jnp.sum(jnp.where(in_seg, mx_ref[...], 0.0), axis=1, keepdims=True)
    ex = jnp.exp(s_ref[...] - mrow)
    out_ref[...] += jnp.sum(jnp.where(in_seg, ex, 0.0), axis=0, keepdims=True)


def _norm_kernel(starts_ref, ends_ref, s_ref, mx_ref, den_ref, out_ref):
    t = pl.program_id(0)
    rows = t * _T + lax.broadcasted_iota(jnp.int32, (_T, 1), 0)
    in_seg = (rows >= starts_ref[...]) & (rows < ends_ref[...])
    mrow = jnp.sum(jnp.where(in_seg, mx_ref[...], 0.0), axis=1, keepdims=True)
    drow = jnp.sum(jnp.where(in_seg, den_ref[...], 0.0), axis=1, keepdims=True)
    out_ref[...] = jnp.exp(s_ref[...] - mrow) / drow


def kernel(prev_hidden_states, encoder_output, tree_sizes, W, b, v):
    w1t = W[:, :_HD].T
    w2t = W[:, _HD:].T
    csum = jnp.cumsum(tree_sizes.astype(jnp.int32))
    starts = jnp.concatenate(
        [jnp.zeros((1,), jnp.int32), csum[:-1]]).reshape(1, _B)
    ends = csum.reshape(1, _B)
    b2 = b.reshape(1, _HE)
    vrow = v.reshape(1, _HE)

    p_tab = pl.pallas_call(
        _p_kernel,
        out_shape=jax.ShapeDtypeStruct((_B, _HE), jnp.float32),
    )(prev_hidden_states, w1t, b2)

    grid = _N // _T
    scores, segmax = pl.pallas_call(
        _scores_kernel,
        grid=(grid,),
        in_specs=[
            pl.BlockSpec((1, _B), lambda t: (0, 0)),
            pl.BlockSpec((1, _B), lambda t: (0, 0)),
            pl.BlockSpec((_T, _HE), lambda t: (t, 0)),
            pl.BlockSpec((_HE, _HE), lambda t: (0, 0)),
            pl.BlockSpec((_B, _HE), lambda t: (0, 0)),
            pl.BlockSpec((1, _HE), lambda t: (0, 0)),
        ],
        out_specs=[
            pl.BlockSpec((_T, 1), lambda t: (t, 0)),
            pl.BlockSpec((1, _B), lambda t: (0, 0)),
        ],
        out_shape=[
            jax.ShapeDtypeStruct((_N, 1), jnp.float32),
            jax.ShapeDtypeStruct((1, _B), jnp.float32),
        ],
    )(starts, ends, encoder_output, w2t, p_tab, vrow)

    densum = pl.pallas_call(
        _expsum_kernel,
        grid=(grid,),
        in_specs=[
            pl.BlockSpec((1, _B), lambda t: (0, 0)),
            pl.BlockSpec((1, _B), lambda t: (0, 0)),
            pl.BlockSpec((_T, 1), lambda t: (t, 0)),
            pl.BlockSpec((1, _B), lambda t: (0, 0)),
        ],
        out_specs=pl.BlockSpec((1, _B), lambda t: (0, 0)),
        out_shape=jax.ShapeDtypeStruct((1, _B), jnp.float32),
    )(starts, ends, scores, segmax)

    att = pl.pallas_call(
        _norm_kernel,
        grid=(grid,),
        in_specs=[
            pl.BlockSpec((1, _B), lambda t: (0, 0)),
            pl.BlockSpec((1, _B), lambda t: (0, 0)),
            pl.BlockSpec((_T, 1), lambda t: (t, 0)),
            pl.BlockSpec((1, _B), lambda t: (0, 0)),
            pl.BlockSpec((1, _B), lambda t: (0, 0)),
        ],
        out_specs=pl.BlockSpec((_T, 1), lambda t: (t, 0)),
        out_shape=jax.ShapeDtypeStruct((_N, 1), jnp.float32),
    )(starts, ends, scores, segmax, densum)

    return att
